# Initial kernel scaffold; baseline (speedup 1.0000x reference)
import jax
import jax.numpy as jnp
from jax.experimental import pallas as pl


def kernel(x, edge_index, Wl1, bl1, Wr1, Wl2, bl2, Wr2):
    raise NotImplementedError("write your pallas kernel here")


# speedup vs baseline: 4.3863x; 4.3863x over previous
import functools

import jax
import jax.numpy as jnp
from jax import lax
from jax.experimental import pallas as pl
from jax.experimental.pallas import tpu as pltpu
from jax.experimental.pallas import tpu_sc as plsc

N = 10000
E = 320000
D = 128

NC = 2
NS = 16
NW = NC * NS

C = 128
EPW = E // NW
G = (EPW + C - 1) // C
PE = NW * G * C

ROWS_PT = (-(-(N + 1) // NS) + 7) // 8 * 8
NP_ = ROWS_PT * NS


def _sc_agg_body(x_hbm, srcp_hbm, dstp_hbm, agg_out, cnt_out,
                 src_v, dst_v, rows_v, ones_v, z16_v, agg_sp, cnt_sp,
                 with_cnt):
    c = lax.axis_index("c")
    s = lax.axis_index("s")
    wid = c * NS + s

    zero = jnp.zeros((16,), jnp.float32)
    one = jnp.ones((16,), jnp.float32)

    def zrow(r, _):
        for k in range(D // 16):
            rows_v[r, pl.ds(k * 16, 16)] = zero
        return 0
    lax.fori_loop(0, C, zrow, 0)

    if with_cnt:
        def zrow16(r, _):
            ones_v[r, pl.ds(0, 16)] = one
            return 0
        lax.fori_loop(0, C, zrow16, 0)
        for r in range(8):
            z16_v[r, pl.ds(0, 16)] = zero

    base = s * ROWS_PT
    n_full = ROWS_PT // C
    for j in range(n_full):
        pltpu.sync_copy(rows_v, agg_sp.at[pl.ds(base + j * C, C)])
    rem = ROWS_PT - n_full * C
    if rem > 0:
        pltpu.sync_copy(rows_v.at[pl.ds(0, rem)],
                        agg_sp.at[pl.ds(base + n_full * C, rem)])
    if with_cnt:
        def zcnt(j, _):
            pltpu.sync_copy(z16_v, cnt_sp.at[pl.ds(base + j * 8, 8)])
            return 0
        lax.fori_loop(0, ROWS_PT // 8, zcnt, 0)

    plsc.subcore_barrier()

    def chunk(g, _):
        pltpu.sync_copy(srcp_hbm.at[wid, g], src_v.at[0])
        pltpu.sync_copy(dstp_hbm.at[wid, g], dst_v.at[0])
        pltpu.sync_copy(x_hbm.at[src_v.at[0]], rows_v)
        pltpu.sync_copy(rows_v, agg_sp.at[dst_v.at[0]], add=True)
        if with_cnt:
            pltpu.sync_copy(ones_v, cnt_sp.at[dst_v.at[0]], add=True)
        return 0
    lax.fori_loop(0, G, chunk, 0)

    plsc.subcore_barrier()

    pltpu.sync_copy(agg_sp.at[pl.ds(base, ROWS_PT)],
                    agg_out.at[c, pl.ds(base, ROWS_PT)])
    if with_cnt:
        pltpu.sync_copy(cnt_sp.at[pl.ds(base, ROWS_PT)],
                        cnt_out.at[c, pl.ds(base, ROWS_PT)])


def _make_sc_agg(with_cnt):
    mesh = plsc.VectorSubcoreMesh(core_axis_name="c", subcore_axis_name="s")
    out_type = [jax.ShapeDtypeStruct((NC, NP_, D), jnp.float32)]
    if with_cnt:
        out_type.append(jax.ShapeDtypeStruct((NC, NP_, 16), jnp.float32))
    scratch = [
        pltpu.VMEM((1, C), jnp.int32),
        pltpu.VMEM((1, C), jnp.int32),
        pltpu.VMEM((C, D), jnp.float32),
        pltpu.VMEM((C, 16), jnp.float32),
        pltpu.VMEM((8, 16), jnp.float32),
        pltpu.VMEM_SHARED((NP_, D), jnp.float32),
        pltpu.VMEM_SHARED((NP_, 16), jnp.float32),
    ]

    if with_cnt:
        def body(x_hbm, srcp, dstp, agg_out, cnt_out,
                 src_v, dst_v, rows_v, ones_v, z16_v, agg_sp, cnt_sp):
            _sc_agg_body(x_hbm, srcp, dstp, agg_out, cnt_out,
                         src_v, dst_v, rows_v, ones_v, z16_v, agg_sp, cnt_sp,
                         True)
    else:
        def body(x_hbm, srcp, dstp, agg_out,
                 src_v, dst_v, rows_v, ones_v, z16_v, agg_sp, cnt_sp):
            _sc_agg_body(x_hbm, srcp, dstp, agg_out, None,
                         src_v, dst_v, rows_v, ones_v, z16_v, agg_sp, cnt_sp,
                         False)

    return pl.kernel(body, out_type=out_type, mesh=mesh,
                     scratch_types=scratch,
                     compiler_params=pltpu.CompilerParams(
                         use_tc_tiling_on_sc=False))


_sc_agg_cnt = _make_sc_agg(True)
_sc_agg = _make_sc_agg(False)

BLK = 1000
_DN = (((1,), (1,)), ((), ()))


def _tc_layer1_body(aggp, cntp, x, wl, bl, wr, h_out):
    agg = aggp[0] + aggp[1]
    cnt = cntp[0, :, 0:1] + cntp[1, :, 0:1]
    mean = agg / jnp.maximum(cnt, 1.0)
    h = (lax.dot_general(mean, wl[...], _DN,
                         precision=lax.Precision.HIGHEST)
         + lax.dot_general(x[...], wr[...], _DN,
                           precision=lax.Precision.HIGHEST)
         + bl[...])
    h_out[...] = jnp.maximum(h, 0.0)


def _tc_layer2_body(aggp, cntp, h, wl, bl, wr, out):
    agg = aggp[0] + aggp[1]
    cnt = cntp[0, :, 0:1] + cntp[1, :, 0:1]
    mean = agg / jnp.maximum(cnt, 1.0)
    o = (lax.dot_general(mean, wl[...], _DN,
                         precision=lax.Precision.HIGHEST)
         + lax.dot_general(h[...], wr[...], _DN,
                           precision=lax.Precision.HIGHEST)
         + bl[...])
    norm = jnp.sqrt(jnp.sum(o * o, axis=1, keepdims=True))
    out[...] = o / jnp.maximum(norm, 1e-12)


def _tc_layer(body):
    return pl.pallas_call(
        body,
        grid=(N // BLK,),
        in_specs=[
            pl.BlockSpec((NC, BLK, D), lambda i: (0, i, 0)),
            pl.BlockSpec((NC, BLK, 16), lambda i: (0, i, 0)),
            pl.BlockSpec((BLK, D), lambda i: (i, 0)),
            pl.BlockSpec((D, D), lambda i: (0, 0)),
            pl.BlockSpec((1, D), lambda i: (0, 0)),
            pl.BlockSpec((D, D), lambda i: (0, 0)),
        ],
        out_specs=pl.BlockSpec((BLK, D), lambda i: (i, 0)),
        out_shape=jax.ShapeDtypeStruct((N, D), jnp.float32),
    )


_tc_layer1 = _tc_layer(_tc_layer1_body)
_tc_layer2 = _tc_layer(_tc_layer2_body)


def kernel(x, edge_index, Wl1, bl1, Wr1, Wl2, bl2, Wr2):
    src = edge_index[0]
    dst = edge_index[1]
    pad = PE - E
    srcp = jnp.concatenate(
        [src, jnp.zeros((pad,), jnp.int32)]).reshape(NW, G, C)
    dstp = jnp.concatenate(
        [dst, jnp.full((pad,), N, jnp.int32)]).reshape(NW, G, C)

    agg1, cnt = _sc_agg_cnt(x, srcp, dstp)
    h = _tc_layer1(agg1, cnt, x, Wl1, bl1.reshape(1, D), Wr1)
    (agg2,) = _sc_agg(h, srcp, dstp)
    out = _tc_layer2(agg2, cnt, h, Wl2, bl2.reshape(1, D), Wr2)
    return out
